# baseline (speedup 1.0000x reference)
import functools

import jax
import jax.numpy as jnp
from jax import lax
from jax.experimental import pallas as pl
from jax.experimental.pallas import tpu as pltpu
from jax.experimental.pallas import tpu_sc as plsc

NUM_TABLES = 26
DIM = 32
VOCAB = 100000
BATCH = 16384
LANES = 128
TBLK = 4096
NROW = 102400
CH = 256
NBUF = 3
GROUPS = (7, 7, 6, 6)


def _pad_kernel(x_ref, o_ref):
    o_ref[:, :DIM] = x_ref[...].T


_padT = pl.pallas_call(
    _pad_kernel,
    grid=(NROW // TBLK,),
    in_specs=[pl.BlockSpec((DIM, TBLK), lambda j: (0, j))],
    out_specs=pl.BlockSpec((TBLK, LANES), lambda j: (j, 0)),
    out_shape=jax.ShapeDtypeStruct((NROW, LANES), jnp.float32),
)


@functools.cache
def _build_group(nt):
    info = plsc.get_sparse_core_info()
    nc, ns = info.num_cores, info.num_subcores
    nw = nc * ns
    bpw = BATCH // nw
    nch = bpw // CH
    mesh = plsc.VectorSubcoreMesh(core_axis_name="c", subcore_axis_name="s")

    @functools.partial(
        pl.kernel,
        mesh=mesh,
        out_type=jax.ShapeDtypeStruct((nt * BATCH, LANES), jnp.float32),
        scratch_types=(
            [pltpu.VMEM((bpw,), jnp.int32) for _ in range(nt)]
            + [pltpu.VMEM((CH, LANES), jnp.float32) for _ in range(NBUF)]
            + [pltpu.SemaphoreType.DMA for _ in range(2 * NBUF + 1)]
        ),
        compiler_params=pltpu.CompilerParams(use_tc_tiling_on_sc=True),
    )
    def k(*refs):
        idx_refs = refs[:nt]
        tab_refs = refs[nt:2 * nt]
        out = refs[2 * nt]
        rest = refs[2 * nt + 1:]
        idx_vm = rest[:nt]
        bufs = rest[nt:nt + NBUF]
        gsems = rest[nt + NBUF:nt + 2 * NBUF]
        wsems = rest[nt + 2 * NBUF:nt + 3 * NBUF]
        isem = rest[nt + 3 * NBUF]

        wid = lax.axis_index("s") * nc + lax.axis_index("c")
        base = wid * bpw

        idescs = [
            pltpu.async_copy(idx_refs[t].at[pl.ds(base, bpw)], idx_vm[t],
                             isem)
            for t in range(nt)
        ]
        for d in idescs:
            d.wait()

        chunks = [(t, h) for t in range(nt) for h in range(nch)]

        def gather(c, s):
            t, h = chunks[c]
            return pltpu.async_copy(
                tab_refs[t].at[idx_vm[t].at[pl.ds(h * CH, CH)]],
                bufs[s], gsems[s])

        def write(c, s):
            t, h = chunks[c]
            return pltpu.async_copy(
                bufs[s],
                out.at[pl.ds(t * BATCH + base + h * CH, CH)],
                wsems[s])

        n = len(chunks)
        gd = [None] * NBUF
        wd = [None] * NBUF
        for c in range(min(NBUF, n)):
            gd[c % NBUF] = gather(c, c % NBUF)
        for c in range(n):
            s = c % NBUF
            gd[s].wait()
            wd[s] = write(c, s)
            nxt = c + NBUF
            if nxt < n:
                wd[s].wait()
                wd[s] = None
                gd[s] = gather(nxt, s)
        for s in range(NBUF):
            if wd[s] is not None:
                wd[s].wait()

    return k


def kernel(f0, f1, f2, f3, f4, f5, f6, f7, f8, f9, f10, f11, f12, f13, f14,
           f15, f16, f17, f18, f19, f20, f21, f22, f23, f24, f25,
           W_f0, W_f1, W_f2, W_f3, W_f4, W_f5, W_f6, W_f7, W_f8, W_f9, W_f10,
           W_f11, W_f12, W_f13, W_f14, W_f15, W_f16, W_f17, W_f18, W_f19,
           W_f20, W_f21, W_f22, W_f23, W_f24, W_f25):
    idx = [f0, f1, f2, f3, f4, f5, f6, f7, f8, f9, f10, f11, f12, f13, f14,
           f15, f16, f17, f18, f19, f20, f21, f22, f23, f24, f25]
    tabs = [W_f0, W_f1, W_f2, W_f3, W_f4, W_f5, W_f6, W_f7, W_f8, W_f9,
            W_f10, W_f11, W_f12, W_f13, W_f14, W_f15, W_f16, W_f17, W_f18,
            W_f19, W_f20, W_f21, W_f22, W_f23, W_f24, W_f25]
    pieces = []
    off = 0
    for nt in GROUPS:
        gtabs = [_padT(w.T) for w in tabs[off:off + nt]]
        pieces.append(_build_group(nt)(*idx[off:off + nt], *gtabs))
        off += nt
    full = jnp.concatenate(pieces, axis=0)
    emb = full.reshape(NUM_TABLES, BATCH, LANES)[:, :, :DIM]
    return emb.transpose(1, 0, 2).reshape(BATCH, NUM_TABLES * DIM)


# speedup vs baseline: 1.0178x; 1.0178x over previous
import functools

import jax
import jax.numpy as jnp
from jax import lax
from jax.experimental import pallas as pl
from jax.experimental.pallas import tpu as pltpu
from jax.experimental.pallas import tpu_sc as plsc

NUM_TABLES = 26
DIM = 32
VOCAB = 100000
BATCH = 16384
LANES = 128
TBLK = 4096
NROW = 102400
CH = 256
NBUF = 3
GROUPS = (2,) * 13


def _pad_kernel(x_ref, o_ref):
    o_ref[:, :DIM] = x_ref[...].T


_padT = pl.pallas_call(
    _pad_kernel,
    grid=(NROW // TBLK,),
    in_specs=[pl.BlockSpec((DIM, TBLK), lambda j: (0, j))],
    out_specs=pl.BlockSpec((TBLK, LANES), lambda j: (j, 0)),
    out_shape=jax.ShapeDtypeStruct((NROW, LANES), jnp.float32),
)


@functools.cache
def _build_group(nt):
    info = plsc.get_sparse_core_info()
    nc, ns = info.num_cores, info.num_subcores
    nw = nc * ns
    bpw = BATCH // nw
    nch = bpw // CH
    mesh = plsc.VectorSubcoreMesh(core_axis_name="c", subcore_axis_name="s")

    @functools.partial(
        pl.kernel,
        mesh=mesh,
        out_type=jax.ShapeDtypeStruct((nt * BATCH, LANES), jnp.float32),
        scratch_types=(
            [pltpu.VMEM((bpw,), jnp.int32) for _ in range(nt)]
            + [pltpu.VMEM((CH, LANES), jnp.float32) for _ in range(NBUF)]
            + [pltpu.SemaphoreType.DMA for _ in range(2 * NBUF + 1)]
        ),
        compiler_params=pltpu.CompilerParams(use_tc_tiling_on_sc=True),
    )
    def k(*refs):
        idx_refs = refs[:nt]
        tab_refs = refs[nt:2 * nt]
        out = refs[2 * nt]
        rest = refs[2 * nt + 1:]
        idx_vm = rest[:nt]
        bufs = rest[nt:nt + NBUF]
        gsems = rest[nt + NBUF:nt + 2 * NBUF]
        wsems = rest[nt + 2 * NBUF:nt + 3 * NBUF]
        isem = rest[nt + 3 * NBUF]

        wid = lax.axis_index("s") * nc + lax.axis_index("c")
        base = wid * bpw

        idescs = [
            pltpu.async_copy(idx_refs[t].at[pl.ds(base, bpw)], idx_vm[t],
                             isem)
            for t in range(nt)
        ]
        for d in idescs:
            d.wait()

        chunks = [(t, h) for t in range(nt) for h in range(nch)]

        def gather(c, s):
            t, h = chunks[c]
            return pltpu.async_copy(
                tab_refs[t].at[idx_vm[t].at[pl.ds(h * CH, CH)]],
                bufs[s], gsems[s])

        def write(c, s):
            t, h = chunks[c]
            return pltpu.async_copy(
                bufs[s],
                out.at[pl.ds(t * BATCH + base + h * CH, CH)],
                wsems[s])

        n = len(chunks)
        gd = [None] * NBUF
        wd = [None] * NBUF
        for c in range(min(NBUF, n)):
            gd[c % NBUF] = gather(c, c % NBUF)
        for c in range(n):
            s = c % NBUF
            gd[s].wait()
            wd[s] = write(c, s)
            nxt = c + NBUF
            if nxt < n:
                wd[s].wait()
                wd[s] = None
                gd[s] = gather(nxt, s)
        for s in range(NBUF):
            if wd[s] is not None:
                wd[s].wait()

    return k


def kernel(f0, f1, f2, f3, f4, f5, f6, f7, f8, f9, f10, f11, f12, f13, f14,
           f15, f16, f17, f18, f19, f20, f21, f22, f23, f24, f25,
           W_f0, W_f1, W_f2, W_f3, W_f4, W_f5, W_f6, W_f7, W_f8, W_f9, W_f10,
           W_f11, W_f12, W_f13, W_f14, W_f15, W_f16, W_f17, W_f18, W_f19,
           W_f20, W_f21, W_f22, W_f23, W_f24, W_f25):
    idx = [f0, f1, f2, f3, f4, f5, f6, f7, f8, f9, f10, f11, f12, f13, f14,
           f15, f16, f17, f18, f19, f20, f21, f22, f23, f24, f25]
    tabs = [W_f0, W_f1, W_f2, W_f3, W_f4, W_f5, W_f6, W_f7, W_f8, W_f9,
            W_f10, W_f11, W_f12, W_f13, W_f14, W_f15, W_f16, W_f17, W_f18,
            W_f19, W_f20, W_f21, W_f22, W_f23, W_f24, W_f25]
    pieces = []
    off = 0
    for nt in GROUPS:
        gtabs = [_padT(w.T) for w in tabs[off:off + nt]]
        pieces.append(_build_group(nt)(*idx[off:off + nt], *gtabs))
        off += nt
    full = jnp.concatenate(pieces, axis=0)
    emb = full.reshape(NUM_TABLES, BATCH, LANES)[:, :, :DIM]
    return emb.transpose(1, 0, 2).reshape(BATCH, NUM_TABLES * DIM)


# speedup vs baseline: 1.2655x; 1.2434x over previous
import functools

import jax
import jax.numpy as jnp
from jax import lax
from jax.experimental import pallas as pl
from jax.experimental.pallas import tpu as pltpu
from jax.experimental.pallas import tpu_sc as plsc

NUM_TABLES = 26
DIM = 32
VOCAB = 100000
BATCH = 16384
LANES = 128
PACK = LANES // DIM
TBLK = 4096
NROW = 102400
CH = 256
NBUF = 3
GROUPS = (4, 4, 4, 4, 4, 4, 2)


def _pack_kernel(*refs):
    xs, o_ref = refs[:-1], refs[-1]
    for a, x_ref in enumerate(xs):
        o_ref[:, a * DIM:(a + 1) * DIM] = x_ref[...].T


@functools.cache
def _make_pack(nt):
    return pl.pallas_call(
        _pack_kernel,
        grid=(NROW // TBLK,),
        in_specs=[pl.BlockSpec((DIM, TBLK), lambda j: (0, j))
                  for _ in range(nt)],
        out_specs=pl.BlockSpec((TBLK, LANES), lambda j: (j, 0)),
        out_shape=jax.ShapeDtypeStruct((NROW, LANES), jnp.float32),
    )


@functools.cache
def _build_group(nt):
    info = plsc.get_sparse_core_info()
    nc, ns = info.num_cores, info.num_subcores
    nw = nc * ns
    bpw = BATCH // nw
    nch = bpw // CH
    mesh = plsc.VectorSubcoreMesh(core_axis_name="c", subcore_axis_name="s")

    @functools.partial(
        pl.kernel,
        mesh=mesh,
        out_type=jax.ShapeDtypeStruct((nt * BATCH, LANES), jnp.float32),
        scratch_types=(
            [pltpu.VMEM((bpw,), jnp.int32) for _ in range(nt)]
            + [pltpu.VMEM((CH, LANES), jnp.float32) for _ in range(NBUF)]
            + [pltpu.SemaphoreType.DMA for _ in range(2 * NBUF + 1)]
        ),
        compiler_params=pltpu.CompilerParams(use_tc_tiling_on_sc=True),
    )
    def k(*refs):
        idx_refs = refs[:nt]
        pack = refs[nt]
        out = refs[nt + 1]
        rest = refs[nt + 2:]
        idx_vm = rest[:nt]
        bufs = rest[nt:nt + NBUF]
        gsems = rest[nt + NBUF:nt + 2 * NBUF]
        wsems = rest[nt + 2 * NBUF:nt + 3 * NBUF]
        isem = rest[nt + 3 * NBUF]

        wid = lax.axis_index("s") * nc + lax.axis_index("c")
        base = wid * bpw

        idescs = [
            pltpu.async_copy(idx_refs[t].at[pl.ds(base, bpw)], idx_vm[t],
                             isem)
            for t in range(nt)
        ]
        for d in idescs:
            d.wait()

        chunks = [(t, h) for t in range(nt) for h in range(nch)]

        def gather(c, s):
            t, h = chunks[c]
            return pltpu.async_copy(
                pack.at[idx_vm[t].at[pl.ds(h * CH, CH)]],
                bufs[s], gsems[s])

        def write(c, s):
            t, h = chunks[c]
            return pltpu.async_copy(
                bufs[s],
                out.at[pl.ds(t * BATCH + base + h * CH, CH)],
                wsems[s])

        n = len(chunks)
        gd = [None] * NBUF
        wd = [None] * NBUF
        for c in range(min(NBUF, n)):
            gd[c % NBUF] = gather(c, c % NBUF)
        for c in range(n):
            s = c % NBUF
            gd[s].wait()
            wd[s] = write(c, s)
            nxt = c + NBUF
            if nxt < n:
                wd[s].wait()
                wd[s] = None
                gd[s] = gather(nxt, s)
        for s in range(NBUF):
            if wd[s] is not None:
                wd[s].wait()

    return k


def kernel(f0, f1, f2, f3, f4, f5, f6, f7, f8, f9, f10, f11, f12, f13, f14,
           f15, f16, f17, f18, f19, f20, f21, f22, f23, f24, f25,
           W_f0, W_f1, W_f2, W_f3, W_f4, W_f5, W_f6, W_f7, W_f8, W_f9, W_f10,
           W_f11, W_f12, W_f13, W_f14, W_f15, W_f16, W_f17, W_f18, W_f19,
           W_f20, W_f21, W_f22, W_f23, W_f24, W_f25):
    idx = [f0, f1, f2, f3, f4, f5, f6, f7, f8, f9, f10, f11, f12, f13, f14,
           f15, f16, f17, f18, f19, f20, f21, f22, f23, f24, f25]
    tabs = [W_f0, W_f1, W_f2, W_f3, W_f4, W_f5, W_f6, W_f7, W_f8, W_f9,
            W_f10, W_f11, W_f12, W_f13, W_f14, W_f15, W_f16, W_f17, W_f18,
            W_f19, W_f20, W_f21, W_f22, W_f23, W_f24, W_f25]
    cols = []
    off = 0
    for nt in GROUPS:
        pack = _make_pack(nt)(*[w.T for w in tabs[off:off + nt]])
        piece = _build_group(nt)(*idx[off:off + nt], pack)
        p3 = piece.reshape(nt, BATCH, LANES)
        for a in range(nt):
            cols.append(p3[a, :, a * DIM:(a + 1) * DIM])
        off += nt
    return jnp.concatenate(cols, axis=-1)
